# baseline (speedup 1.0000x reference)
import jax
import jax.numpy as jnp
from jax import lax
from jax.experimental import pallas as pl
from jax.experimental.pallas import tpu as pltpu
from jax.experimental.pallas import tpu_sc as plsc

NC, NS, L = 2, 16, 16
NW = NC * NS
B, S, H = 4, 8192, 1024
PPW = S // NW
C = 32
NJ = PPW // C


def _sc_body(ids_hbm, wte_hbm, wpe_hbm, out_hbm, idx_v, wpe_v, rows_v,
             sem_g, sem_o):
    w = lax.axis_index("s") * NC + lax.axis_index("c")
    pos0 = w * PPW
    pltpu.sync_copy(ids_hbm.at[w], idx_v)

    def j_body(j, _):
        pltpu.sync_copy(wpe_hbm.at[pl.ds(pos0 + j * C, C)], wpe_v)
        for b in range(B):
            pltpu.async_copy(wte_hbm.at[idx_v.at[b * NJ + j]], rows_v,
                             sem_g).wait()

            def r_body(r, _):
                for k in range(H // L):
                    sl = pl.ds(k * L, L)
                    rows_v[r, sl] = rows_v[r, sl] + wpe_v[r, sl]
                return 0

            lax.fori_loop(0, C, r_body, 0)
            pltpu.async_copy(rows_v, out_hbm.at[b, pl.ds(pos0 + j * C, C)],
                             sem_o).wait()
        return 0

    lax.fori_loop(0, NJ, j_body, 0)


def _sc_call(ids_r, wte, wpe):
    mesh = plsc.VectorSubcoreMesh(core_axis_name="c", subcore_axis_name="s",
                                  num_cores=NC, num_subcores=NS)
    f = pl.kernel(
        _sc_body,
        out_type=jax.ShapeDtypeStruct((B, S, H), jnp.float32),
        mesh=mesh,
        scratch_types=[
            pltpu.VMEM((B * NJ, C), jnp.int32),
            pltpu.VMEM((C, H), jnp.float32),
            pltpu.VMEM((C, H), jnp.float32),
            pltpu.SemaphoreType.DMA,
            pltpu.SemaphoreType.DMA,
        ],
    )
    return f(ids_r, wte, wpe)


@jax.jit
def kernel(input_ids, wte, wpe):
    ids = input_ids.astype(jnp.int32)
    ids_r = (ids.reshape(B, NW, NJ, C)
                .transpose(1, 0, 2, 3)
                .reshape(NW, B * NJ, C))
    return _sc_call(ids_r, wte, wpe)


# speedup vs baseline: 1.5559x; 1.5559x over previous
import jax
import jax.numpy as jnp
from jax import lax
from jax.experimental import pallas as pl
from jax.experimental.pallas import tpu as pltpu
from jax.experimental.pallas import tpu_sc as plsc

NC, NS, L = 2, 16, 16
NW = NC * NS
B, S, H = 4, 8192, 1024
PPW = S // NW
C = 16
NJ = PPW // C


def _sc_body(ids_hbm, wte_hbm, wpe_hbm, out_hbm, idx_v, g0, g1, w0, w1,
             sg0, sg1, sw0, sw1, so0, so1):
    w = lax.axis_index("s") * NC + lax.axis_index("c")
    pos0 = w * PPW
    gbuf, wbuf = (g0, g1), (w0, w1)
    sg, sw, so = (sg0, sg1), (sw0, sw1), (so0, so1)

    pltpu.sync_copy(ids_hbm.at[w], idx_v)

    def fire_wpe(j, jp):
        pltpu.async_copy(wpe_hbm.at[pl.ds(pos0 + j * C, C)], wbuf[jp],
                         sw[jp])

    def fire_gather(row, gp):
        pltpu.async_copy(wte_hbm.at[idx_v.at[row]], gbuf[gp], sg[gp])

    def wait_gather(gp):
        pltpu.make_async_copy(wte_hbm.at[idx_v.at[0]], gbuf[gp],
                              sg[gp]).wait()

    def wait_wpe(jp):
        pltpu.make_async_copy(wpe_hbm.at[pl.ds(0, C)], wbuf[jp],
                              sw[jp]).wait()

    def wait_out(gp):
        pltpu.make_async_copy(gbuf[gp], out_hbm.at[0, pl.ds(0, C)],
                              so[gp]).wait()

    def add_rows(gp, jp):
        def r_body(r, _):
            for k in range(H // L):
                sl = pl.ds(k * L, L)
                plsc.addupdate(gbuf[gp].at[r, sl], wbuf[jp][r, sl])
            return 0
        lax.fori_loop(0, C, r_body, 0)

    def do_chunk(j, b, jp, first=False, last=False):
        gp = b % 2
        if b == 0:
            fire_wpe(j + 1 if not last else 0, (jp + 1) % 2)
        if not first:
            wait_out((gp + 1) % 2)
        if b < B - 1:
            nrow = (b + 1) * NJ + j
        else:
            nrow = (j + 1) if not last else 0
        fire_gather(nrow, (gp + 1) % 2)
        wait_gather(gp)
        if b == 0:
            wait_wpe(jp)
        add_rows(gp, jp)
        pltpu.async_copy(gbuf[gp], out_hbm.at[b, pl.ds(pos0 + j * C, C)],
                         so[gp])

    fire_wpe(0, 0)
    fire_gather(0, 0)
    for b in range(B):
        do_chunk(0, b, 0, first=(b == 0))

    def jj_body(jj, _):
        for j2 in range(2):
            for b in range(B):
                do_chunk(1 + 2 * jj + j2, b, (1 + j2) % 2)
        return 0
    lax.fori_loop(0, (NJ - 2) // 2, jj_body, 0)

    for b in range(B):
        do_chunk(NJ - 1, b, (NJ - 1) % 2, last=True)

    wait_gather(0)
    wait_wpe(NJ % 2)
    wait_out(1)


def _sc_call(ids_r, wte, wpe):
    mesh = plsc.VectorSubcoreMesh(core_axis_name="c", subcore_axis_name="s",
                                  num_cores=NC, num_subcores=NS)
    f = pl.kernel(
        _sc_body,
        out_type=jax.ShapeDtypeStruct((B, S, H), jnp.float32),
        mesh=mesh,
        scratch_types=[
            pltpu.VMEM((B * NJ, C), jnp.int32),
            pltpu.VMEM((C, H), jnp.float32),
            pltpu.VMEM((C, H), jnp.float32),
            pltpu.VMEM((C, H), jnp.float32),
            pltpu.VMEM((C, H), jnp.float32),
            pltpu.SemaphoreType.DMA,
            pltpu.SemaphoreType.DMA,
            pltpu.SemaphoreType.DMA,
            pltpu.SemaphoreType.DMA,
            pltpu.SemaphoreType.DMA,
            pltpu.SemaphoreType.DMA,
        ],
    )
    return f(ids_r, wte, wpe)


@jax.jit
def kernel(input_ids, wte, wpe):
    ids = input_ids.astype(jnp.int32)
    ids_r = (ids.reshape(B, NW, NJ, C)
                .transpose(1, 0, 2, 3)
                .reshape(NW, B * NJ, C))
    return _sc_call(ids_r, wte, wpe)


# speedup vs baseline: 1.9641x; 1.2623x over previous
import jax
import jax.numpy as jnp
from jax import lax
from jax.experimental import pallas as pl
from jax.experimental.pallas import tpu as pltpu
from jax.experimental.pallas import tpu_sc as plsc

NC, NS, L = 2, 16, 16
NW = NC * NS
B, S, H = 4, 8192, 1024
PPW = S // NW
C = 8
NJ = PPW // C
K = H // L


def _sc_body(ids_hbm, wte_hbm, wpe_hbm, out_hbm, idx_v,
             ga00, ga01, ga02, ga03, ga10, ga11, ga12, ga13, wb0, wb1,
             sg00, sg01, sg02, sg03, sg10, sg11, sg12, sg13, sw0, sw1,
             so00, so01, so02, so03, so10, so11, so12, so13):
    w = lax.axis_index("s") * NC + lax.axis_index("c")
    pos0 = w * PPW
    ga = ((ga00, ga01, ga02, ga03), (ga10, ga11, ga12, ga13))
    sg = ((sg00, sg01, sg02, sg03), (sg10, sg11, sg12, sg13))
    so = ((so00, so01, so02, so03), (so10, so11, so12, so13))
    wb = (wb0, wb1)
    sw = (sw0, sw1)

    pltpu.sync_copy(ids_hbm.at[w], idx_v)

    def fire_inputs(j, p):
        pltpu.async_copy(wpe_hbm.at[pl.ds(pos0 + j * C, C)], wb[p], sw[p])
        for b in range(B):
            pltpu.async_copy(wte_hbm.at[idx_v.at[b * NJ + j]], ga[p][b],
                             sg[p][b])

    def wait_inputs(p):
        pltpu.make_async_copy(wpe_hbm.at[pl.ds(0, C)], wb[p], sw[p]).wait()
        for b in range(B):
            pltpu.make_async_copy(wte_hbm.at[idx_v.at[0]], ga[p][b],
                                  sg[p][b]).wait()

    def wait_outs(p):
        for b in range(B):
            pltpu.make_async_copy(ga[p][b], out_hbm.at[0, pl.ds(0, C)],
                                  so[p][b]).wait()

    def do_step(j, p, first=False, last=False):
        if not last:
            if not first:
                wait_outs(1 - p)
            fire_inputs(j + 1, 1 - p)
        wait_inputs(p)

        @plsc.parallel_loop(0, C)
        def _(r):
            for k in range(K):
                sl = pl.ds(k * L, L)
                v = wb[p][r, sl]
                for b in range(B):
                    plsc.addupdate(ga[p][b].at[r, sl], v)

        for b in range(B):
            pltpu.async_copy(ga[p][b],
                             out_hbm.at[b, pl.ds(pos0 + j * C, C)],
                             so[p][b])

    fire_inputs(0, 0)
    do_step(0, 0, first=True)

    def jj_body(jj, _):
        for j2 in range(2):
            do_step(1 + 2 * jj + j2, (1 + j2) % 2)
        return 0
    lax.fori_loop(0, (NJ - 2) // 2, jj_body, 0)

    do_step(NJ - 1, (NJ - 1) % 2, last=True)
    wait_outs(0)
    wait_outs(1)


def _sc_call(ids_r, wte, wpe):
    mesh = plsc.VectorSubcoreMesh(core_axis_name="c", subcore_axis_name="s",
                                  num_cores=NC, num_subcores=NS)
    buf = lambda: pltpu.VMEM((C, H), jnp.float32)
    sem = pltpu.SemaphoreType.DMA
    f = pl.kernel(
        _sc_body,
        out_type=jax.ShapeDtypeStruct((B, S, H), jnp.float32),
        mesh=mesh,
        scratch_types=(
            [pltpu.VMEM((B * NJ, C), jnp.int32)]
            + [buf() for _ in range(10)]
            + [sem] * 18
        ),
    )
    return f(ids_r, wte, wpe)


@jax.jit
def kernel(input_ids, wte, wpe):
    ids = input_ids.astype(jnp.int32)
    ids_r = (ids.reshape(B, NW, NJ, C)
                .transpose(1, 0, 2, 3)
                .reshape(NW, B * NJ, C))
    return _sc_call(ids_r, wte, wpe)


# speedup vs baseline: 1.9730x; 1.0045x over previous
import jax
import jax.numpy as jnp
from jax import lax
from jax.experimental import pallas as pl
from jax.experimental.pallas import tpu as pltpu
from jax.experimental.pallas import tpu_sc as plsc

NC, NS, L = 2, 16, 16
NW = NC * NS
B, S, H = 4, 8192, 1024
PPW = S // NW
C = 8
NJ = PPW // C
K = H // L


def _sc_body(ids_hbm, wte_hbm, wpe_hbm, out_hbm, idx_v,
             ga00, ga01, ga02, ga03, ga10, ga11, ga12, ga13, wb0, wb1,
             sg00, sg01, sg02, sg03, sg10, sg11, sg12, sg13, sw0, sw1,
             so00, so01, so02, so03, so10, so11, so12, so13):
    w = lax.axis_index("s") * NC + lax.axis_index("c")
    pos0 = w * PPW
    ga = ((ga00, ga01, ga02, ga03), (ga10, ga11, ga12, ga13))
    sg = ((sg00, sg01, sg02, sg03), (sg10, sg11, sg12, sg13))
    so = ((so00, so01, so02, so03), (so10, so11, so12, so13))
    wb = (wb0, wb1)
    sw = (sw0, sw1)

    pltpu.sync_copy(ids_hbm.at[w], idx_v)

    def fire_inputs(j, p):
        pltpu.async_copy(wpe_hbm.at[pl.ds(pos0 + j * C, C)], wb[p], sw[p])
        for b in range(B):
            pltpu.async_copy(wte_hbm.at[idx_v.at[b * NJ + j]], ga[p][b],
                             sg[p][b])

    def wait_inputs(p):
        pltpu.make_async_copy(wpe_hbm.at[pl.ds(0, C)], wb[p], sw[p]).wait()
        for b in range(B):
            pltpu.make_async_copy(wte_hbm.at[idx_v.at[0]], ga[p][b],
                                  sg[p][b]).wait()

    def wait_outs(p):
        for b in range(B):
            pltpu.make_async_copy(ga[p][b], out_hbm.at[0, pl.ds(0, C)],
                                  so[p][b]).wait()

    def do_step(j, p, first=False, last=False):
        if not last:
            if not first:
                wait_outs(1 - p)
            fire_inputs(j + 1, 1 - p)
        wait_inputs(p)

        @plsc.parallel_loop(0, C)
        def _(r):
            v = wb[p][r, pl.ds(0, L)]
            for k in range(K):
                nv = wb[p][r, pl.ds((k + 1) * L, L)] if k + 1 < K else v
                sl = pl.ds(k * L, L)
                for b in range(B):
                    plsc.addupdate(ga[p][b].at[r, sl], v)
                v = nv

        for b in range(B):
            pltpu.async_copy(ga[p][b],
                             out_hbm.at[b, pl.ds(pos0 + j * C, C)],
                             so[p][b])

    fire_inputs(0, 0)
    do_step(0, 0, first=True)

    def jj_body(jj, _):
        for j2 in range(2):
            do_step(1 + 2 * jj + j2, (1 + j2) % 2)
        return 0
    lax.fori_loop(0, (NJ - 2) // 2, jj_body, 0)

    do_step(NJ - 1, (NJ - 1) % 2, last=True)
    wait_outs(0)
    wait_outs(1)


def _sc_call(ids_r, wte, wpe):
    mesh = plsc.VectorSubcoreMesh(core_axis_name="c", subcore_axis_name="s",
                                  num_cores=NC, num_subcores=NS)
    buf = lambda: pltpu.VMEM((C, H), jnp.float32)
    sem = pltpu.SemaphoreType.DMA
    f = pl.kernel(
        _sc_body,
        out_type=jax.ShapeDtypeStruct((B, S, H), jnp.float32),
        mesh=mesh,
        scratch_types=(
            [pltpu.VMEM((B * NJ, C), jnp.int32)]
            + [buf() for _ in range(10)]
            + [sem] * 18
        ),
    )
    return f(ids_r, wte, wpe)


@jax.jit
def kernel(input_ids, wte, wpe):
    ids = input_ids.astype(jnp.int32)
    ids_r = (ids.reshape(B, NW, NJ, C)
                .transpose(1, 0, 2, 3)
                .reshape(NW, B * NJ, C))
    return _sc_call(ids_r, wte, wpe)
